# baseline (speedup 1.0000x reference)
import jax
import jax.numpy as jnp
from jax.experimental import pallas as pl

D = 1000
ROWS = 4096 * 20
BLOCK_R = 1024
NUM_BLOCKS = ROWS // BLOCK_R


def _onehot_block(x_ref, o_ref):
    idx = x_ref[0, 0, :]
    iota = jax.lax.broadcasted_iota(jnp.int32, (BLOCK_R, D), 1)
    o_ref[...] = jnp.where(idx[:, None] == iota, jnp.float32(5.0), jnp.float32(0.0))


def kernel(x):
    xf = x.reshape(NUM_BLOCKS, 1, BLOCK_R)
    out = pl.pallas_call(
        _onehot_block,
        grid=(NUM_BLOCKS,),
        in_specs=[pl.BlockSpec((1, 1, BLOCK_R), lambda i: (i, 0, 0))],
        out_specs=pl.BlockSpec((BLOCK_R, D), lambda i: (i, 0)),
        out_shape=jax.ShapeDtypeStruct((ROWS, D), jnp.float32),
    )(xf)
    return out.reshape(4096, 20, D)


# speedup vs baseline: 1.5542x; 1.5542x over previous
import jax
import jax.numpy as jnp
from jax.experimental import pallas as pl

D = 1000
N = 4096
T = 20
BLOCK_B = 64
NUM_BLOCKS = N // BLOCK_B


def _onehot_block(x_ref, o_ref):
    idx = x_ref[...]
    iota = jax.lax.broadcasted_iota(jnp.int32, (BLOCK_B, T, D), 2)
    o_ref[...] = jnp.where(idx[:, :, None] == iota, jnp.float32(5.0),
                           jnp.float32(0.0))


def kernel(x):
    return pl.pallas_call(
        _onehot_block,
        grid=(NUM_BLOCKS,),
        in_specs=[pl.BlockSpec((BLOCK_B, T), lambda i: (i, 0))],
        out_specs=pl.BlockSpec((BLOCK_B, T, D), lambda i: (i, 0, 0)),
        out_shape=jax.ShapeDtypeStruct((N, T, D), jnp.float32),
    )(x)


# speedup vs baseline: 1.5706x; 1.0105x over previous
import jax
import jax.numpy as jnp
from jax import lax
from jax.experimental import pallas as pl
from jax.experimental.pallas import tpu as pltpu

D = 1000
N = 4096
T = 20
BB = 64
NUM = N // BB
NBUF = 4


def _body(x_ref, o_hbm, buf, sems):
    i = pl.program_id(0)
    b = lax.rem(i, NBUF)

    @pl.when(i >= NBUF)
    def _():
        j = i - NBUF
        pltpu.make_async_copy(
            buf.at[b], o_hbm.at[pl.ds(j * BB, BB), :, :], sems.at[b]
        ).wait()

    idx = x_ref[...]
    iota = jax.lax.broadcasted_iota(jnp.int32, (BB, T, D), 2)
    buf[b, ...] = jnp.where(idx[:, :, None] == iota, jnp.float32(5.0),
                            jnp.float32(0.0))
    pltpu.make_async_copy(
        buf.at[b], o_hbm.at[pl.ds(i * BB, BB), :, :], sems.at[b]
    ).start()

    @pl.when(i == NUM - 1)
    def _():
        for off in range(NBUF):
            j = NUM - NBUF + off
            pltpu.make_async_copy(
                buf.at[j % NBUF], o_hbm.at[pl.ds(j * BB, BB), :, :],
                sems.at[j % NBUF]
            ).wait()


def kernel(x):
    return pl.pallas_call(
        _body,
        grid=(NUM,),
        in_specs=[pl.BlockSpec((BB, T), lambda i: (i, 0))],
        out_specs=pl.BlockSpec(memory_space=pltpu.MemorySpace.HBM),
        out_shape=jax.ShapeDtypeStruct((N, T, D), jnp.float32),
        scratch_shapes=[
            pltpu.VMEM((NBUF, BB, T, D), jnp.float32),
            pltpu.SemaphoreType.DMA((NBUF,)),
        ],
    )(x)
